# baseline (speedup 1.0000x reference)
import functools

import jax
import jax.numpy as jnp
from jax import lax
from jax.experimental import pallas as pl
from jax.experimental.pallas import tpu as pltpu
from jax.experimental.pallas import tpu_sc as plsc

N = 10000
NP = 10240
E = 160000
D = 256
HALF = 128
G = 64
NCLS = 40
NSUB = 16
CH = 128
NCHUNK = 80
EPT = NCHUNK * CH
EPAD = NSUB * EPT
RPT = NP // NSUB
DLAG = 8


@functools.cache
def _mesh():
    return plsc.VectorSubcoreMesh(
        core_axis_name="c", subcore_axis_name="s",
        num_cores=2, num_subcores=NSUB)


def _deg_body(dst_hbm, deg_hbm, dst_v, ones_v, zero_v, deg_sh, sem):
    c = lax.axis_index("c")
    s = lax.axis_index("s")
    one16 = jnp.full((16,), 1.0, jnp.float32)
    z16 = jnp.zeros((16,), jnp.float32)

    stage = pltpu.async_copy(dst_hbm.at[s], dst_v, sem)

    def fill(r, _):
        for k in range(HALF // 16):
            ones_v[r, pl.ds(k * 16, 16)] = one16
        return 0
    lax.fori_loop(0, CH, fill, 0)

    def zfill(r, _):
        for k in range(HALF // 16):
            zero_v[r, pl.ds(k * 16, 16)] = z16
        return 0
    lax.fori_loop(0, 64, zfill, 0)

    for j in range(RPT // 64):
        pltpu.sync_copy(zero_v, deg_sh.at[pl.ds(s * RPT + j * 64, 64)])
    stage.wait()
    plsc.subcore_barrier()

    def dfire(j):
        pltpu.async_copy(ones_v, deg_sh.at[dst_v.at[j]], sem, add=True)

    def dwait(_j, _):
        pltpu.make_async_copy(ones_v, deg_sh.at[dst_v.at[0]], sem).wait()
        return 0

    for j in range(DLAG):
        dfire(j)

    def chunk(j, _):
        dfire(j + DLAG)
        dwait(j, 0)
        return 0
    lax.fori_loop(0, NCHUNK - DLAG, chunk, 0)
    lax.fori_loop(0, DLAG, dwait, 0)
    plsc.subcore_barrier()

    half_rows = RPT // 2
    base = c * (NP // 2) + s * half_rows
    pltpu.sync_copy(deg_sh.at[pl.ds(base, half_rows)],
                    deg_hbm.at[pl.ds(base, half_rows)])


@functools.cache
def _sc_degree():
    return pl.kernel(
        _deg_body,
        out_type=jax.ShapeDtypeStruct((NP, HALF), jnp.float32),
        mesh=_mesh(),
        scratch_types=[
            pltpu.VMEM((NCHUNK, CH), jnp.int32),
            pltpu.VMEM((CH, HALF), jnp.float32),
            pltpu.VMEM((64, HALF), jnp.float32),
            pltpu.VMEM_SHARED((NP, HALF), jnp.float32),
            pltpu.SemaphoreType.DMA,
        ],
    )


def _agg_body(h_hbm, src_hbm, dst_hbm, agg_hbm, src_v, dst_v, rows_v, agg_sh,
              sem):
    c = lax.axis_index("c")
    s = lax.axis_index("s")
    z16 = jnp.zeros((16,), jnp.float32)

    def fill(r, _):
        for k in range(HALF // 16):
            rows_v[r, pl.ds(k * 16, 16)] = z16
        return 0
    lax.fori_loop(0, CH, fill, 0)

    for j in range(RPT // CH):
        pltpu.sync_copy(rows_v, agg_sh.at[pl.ds(s * RPT + j * CH, CH)])
    plsc.subcore_barrier()

    pltpu.sync_copy(src_hbm.at[s], src_v)
    pltpu.sync_copy(dst_hbm.at[s], dst_v)

    off = c * NP

    def shift(r, _):
        for k in range(CH // 16):
            v = src_v[r, pl.ds(k * 16, 16)]
            src_v[r, pl.ds(k * 16, 16)] = v + off
        return 0
    lax.fori_loop(0, NCHUNK, shift, 0)

    def chunk(j, _):
        pltpu.async_copy(h_hbm.at[src_v.at[j]], rows_v, sem).wait()
        pltpu.sync_copy(rows_v, agg_sh.at[dst_v.at[j]], add=True)
        return 0
    lax.fori_loop(0, NCHUNK, chunk, 0)
    plsc.subcore_barrier()

    pltpu.sync_copy(agg_sh.at[pl.ds(s * RPT, RPT)],
                    agg_hbm.at[c, pl.ds(s * RPT, RPT)])


@functools.cache
def _sc_aggregate():
    return pl.kernel(
        _agg_body,
        out_type=jax.ShapeDtypeStruct((2, NP, HALF), jnp.float32),
        mesh=_mesh(),
        scratch_types=[
            pltpu.VMEM((NCHUNK, CH), jnp.int32),
            pltpu.VMEM((NCHUNK, CH), jnp.int32),
            pltpu.VMEM((CH, HALF), jnp.float32),
            pltpu.VMEM_SHARED((NP, HALF), jnp.float32),
            pltpu.SemaphoreType.DMA,
        ],
    )


def _dot(a, b):
    return lax.dot_general(a, b, (((1,), (0,)), ((), ())),
                           precision=lax.Precision.HIGHEST,
                           preferred_element_type=jnp.float32)


def _layer_body(agg_ref, h_ref, deg_ref, wl_ref, bl_ref, wr_ref, out_ref):
    a = jnp.concatenate([agg_ref[0], agg_ref[1]], axis=1)
    hh = jnp.concatenate([h_ref[0], h_ref[1]], axis=1)
    rdeg = 1.0 / jnp.maximum(deg_ref[...], 1.0)
    scale = jnp.concatenate([rdeg, rdeg], axis=1)
    out = _dot(a * scale, wl_ref[...]) + _dot(hh, wr_ref[...]) + bl_ref[...]
    out = jnp.maximum(out, 0.0)
    out_ref[0] = out[:, :HALF]
    out_ref[1] = out[:, HALF:]


_ROWB = 256
_GRID = NP // _ROWB


def _tc_layer(agg, h, degb, wl, bl, wr):
    return pl.pallas_call(
        _layer_body,
        grid=(_GRID,),
        in_specs=[
            pl.BlockSpec((2, _ROWB, HALF), lambda i: (0, i, 0)),
            pl.BlockSpec((2, _ROWB, HALF), lambda i: (0, i, 0)),
            pl.BlockSpec((_ROWB, HALF), lambda i: (i, 0)),
            pl.BlockSpec((D, D), lambda i: (0, 0)),
            pl.BlockSpec((1, D), lambda i: (0, 0)),
            pl.BlockSpec((D, D), lambda i: (0, 0)),
        ],
        out_specs=pl.BlockSpec((2, _ROWB, HALF), lambda i: (0, i, 0)),
        out_shape=jax.ShapeDtypeStruct((2, NP, HALF), jnp.float32),
    )(agg, h, degb, wl, bl, wr)


def _pool_body(h_ref, b_ref, w1_ref, b1_ref, w2_ref, b2_ref, out_ref,
               acc_ref, cnt_ref):
    i = pl.program_id(0)

    @pl.when(i == 0)
    def _():
        acc_ref[...] = jnp.zeros_like(acc_ref)
        cnt_ref[...] = jnp.zeros_like(cnt_ref)

    hh = jnp.concatenate([h_ref[0], h_ref[1]], axis=1)
    gids = lax.broadcasted_iota(jnp.int32, (G, _ROWB), 0)
    mask = (jnp.broadcast_to(b_ref[0], (G, _ROWB)) == gids)
    mask = mask.astype(jnp.float32)
    acc_ref[...] += _dot(mask, hh)
    rs = jnp.sum(mask, axis=1, keepdims=True)
    cnt_ref[...] += jnp.broadcast_to(rs, (G, D))

    @pl.when(i == _GRID - 1)
    def _():
        pooled = acc_ref[...] / jnp.maximum(cnt_ref[...], 1.0)
        z = jnp.maximum(_dot(pooled, w1_ref[...]) + b1_ref[...], 0.0)
        logits = _dot(z, w2_ref[...]) + b2_ref[...]
        m = jnp.max(logits, axis=1, keepdims=True)
        lse = jnp.log(jnp.sum(jnp.exp(logits - m), axis=1, keepdims=True)) + m
        out_ref[...] = logits - lse


def _tc_pool_head(h, batch2d, w1, b1, w2p, b2p):
    return pl.pallas_call(
        _pool_body,
        grid=(_GRID,),
        in_specs=[
            pl.BlockSpec((2, _ROWB, HALF), lambda i: (0, i, 0)),
            pl.BlockSpec((1, 1, _ROWB), lambda i: (i, 0, 0)),
            pl.BlockSpec((D, D), lambda i: (0, 0)),
            pl.BlockSpec((1, D), lambda i: (0, 0)),
            pl.BlockSpec((D, HALF), lambda i: (0, 0)),
            pl.BlockSpec((1, HALF), lambda i: (0, 0)),
        ],
        out_specs=pl.BlockSpec((G, HALF), lambda i: (0, 0)),
        out_shape=jax.ShapeDtypeStruct((G, HALF), jnp.float32),
        scratch_shapes=[
            pltpu.VMEM((G, D), jnp.float32),
            pltpu.VMEM((G, D), jnp.float32),
        ],
    )(h, batch2d, w1, b1, w2p, b2p)


@jax.jit
def kernel(x, edge_index, batch, Wl1, bl1, Wr1, Wl2, bl2, Wr2, Wl3, bl3, Wr3,
           W_lin1, b_lin1, W_lin2, b_lin2):
    src = edge_index[0].astype(jnp.int32)
    dst = edge_index[1].astype(jnp.int32)
    src_t = jnp.concatenate(
        [src, jnp.zeros((EPAD - E,), jnp.int32)]).reshape(NSUB, NCHUNK, CH)
    dst_t = jnp.concatenate(
        [dst, jnp.full((EPAD - E,), N, jnp.int32)]).reshape(NSUB, NCHUNK, CH)

    deg = _sc_degree()(dst_t)
    degb = jnp.broadcast_to(deg[:, :1], (NP, HALF))

    xp = jnp.concatenate([x, jnp.zeros((NP - N, D), jnp.float32)])
    h = jnp.stack([xp[:, :HALF], xp[:, HALF:]])

    for wl, bl, wr in ((Wl1, bl1, Wr1), (Wl2, bl2, Wr2), (Wl3, bl3, Wr3)):
        agg = _sc_aggregate()(h.reshape(2 * NP, HALF), src_t, dst_t)
        h = _tc_layer(agg, h, degb, wl, bl.reshape(1, D), wr)

    batch2d = jnp.concatenate(
        [batch.astype(jnp.int32), jnp.full((NP - N,), G, jnp.int32)]
    ).reshape(_GRID, 1, _ROWB)
    w2p = jnp.zeros((D, HALF), jnp.float32).at[:, :NCLS].set(W_lin2)
    b2p = jnp.full((HALF,), -1e30, jnp.float32).at[:NCLS].set(b_lin2)

    out = _tc_pool_head(h, batch2d, W_lin1, b_lin1.reshape(1, D),
                        w2p, b2p.reshape(1, HALF))
    return out[:, :NCLS]


# speedup vs baseline: 1.3160x; 1.3160x over previous
import functools

import jax
import jax.numpy as jnp
from jax import lax
from jax.experimental import pallas as pl
from jax.experimental.pallas import tpu as pltpu
from jax.experimental.pallas import tpu_sc as plsc

N = 10000
NP = 10240
E = 160000
D = 256
HALF = 128
G = 64
NCLS = 40
NSUB = 16
CH = 128
NCHUNK = 79
EPT = NCHUNK * CH
EPAD = NSUB * EPT
RPT = NP // NSUB
DLAG = 8


@functools.cache
def _mesh():
    return plsc.VectorSubcoreMesh(
        core_axis_name="c", subcore_axis_name="s",
        num_cores=2, num_subcores=NSUB)


def _deg_body(dst_hbm, deg_hbm, dst_v, ones_v, zero_v, deg_sh, sem):
    c = lax.axis_index("c")
    s = lax.axis_index("s")
    one16 = jnp.full((16,), 1.0, jnp.float32)
    z16 = jnp.zeros((16,), jnp.float32)

    stage = pltpu.async_copy(dst_hbm.at[s], dst_v, sem)

    def fill(r, _):
        for k in range(HALF // 16):
            ones_v[r, pl.ds(k * 16, 16)] = one16
        return 0
    lax.fori_loop(0, CH, fill, 0)

    def zfill(r, _):
        for k in range(HALF // 16):
            zero_v[r, pl.ds(k * 16, 16)] = z16
        return 0
    lax.fori_loop(0, 64, zfill, 0)

    for j in range(RPT // 64):
        pltpu.sync_copy(zero_v, deg_sh.at[pl.ds(s * RPT + j * 64, 64)])
    stage.wait()
    plsc.subcore_barrier()

    def chunk(j, _):
        pltpu.sync_copy(ones_v, deg_sh.at[dst_v.at[j]], add=True)
        return 0
    lax.fori_loop(0, NCHUNK, chunk, 0)
    plsc.subcore_barrier()

    half_rows = RPT // 2
    base = c * (NP // 2) + s * half_rows
    pltpu.sync_copy(deg_sh.at[pl.ds(base, half_rows)],
                    deg_hbm.at[pl.ds(base, half_rows)])


@functools.cache
def _sc_degree():
    return pl.kernel(
        _deg_body,
        out_type=jax.ShapeDtypeStruct((NP, HALF), jnp.float32),
        mesh=_mesh(),
        scratch_types=[
            pltpu.VMEM((NCHUNK, CH), jnp.int32),
            pltpu.VMEM((CH, HALF), jnp.float32),
            pltpu.VMEM((64, HALF), jnp.float32),
            pltpu.VMEM_SHARED((NP, HALF), jnp.float32),
            pltpu.SemaphoreType.DMA,
        ],
    )


def _agg_body(h_hbm, src_hbm, dst_hbm, agg_hbm, src_v, dst_v, rows_v, agg_sh,
              sem):
    c = lax.axis_index("c")
    s = lax.axis_index("s")
    z16 = jnp.zeros((16,), jnp.float32)

    def fill(r, _):
        for k in range(HALF // 16):
            rows_v[r, pl.ds(k * 16, 16)] = z16
        return 0
    lax.fori_loop(0, CH, fill, 0)

    for j in range(RPT // CH):
        pltpu.sync_copy(rows_v, agg_sh.at[pl.ds(s * RPT + j * CH, CH)])
    plsc.subcore_barrier()

    pltpu.sync_copy(src_hbm.at[s], src_v)
    pltpu.sync_copy(dst_hbm.at[s], dst_v)

    off = c * NP

    def shift(r, _):
        for k in range(CH // 16):
            v = src_v[r, pl.ds(k * 16, 16)]
            src_v[r, pl.ds(k * 16, 16)] = v + off
        return 0
    lax.fori_loop(0, NCHUNK, shift, 0)

    def chunk(j, _):
        pltpu.async_copy(h_hbm.at[src_v.at[j]], rows_v, sem).wait()
        pltpu.sync_copy(rows_v, agg_sh.at[dst_v.at[j]], add=True)
        return 0
    lax.fori_loop(0, NCHUNK, chunk, 0)
    plsc.subcore_barrier()

    pltpu.sync_copy(agg_sh.at[pl.ds(s * RPT, RPT)],
                    agg_hbm.at[c, pl.ds(s * RPT, RPT)])


@functools.cache
def _sc_aggregate():
    return pl.kernel(
        _agg_body,
        out_type=jax.ShapeDtypeStruct((2, NP, HALF), jnp.float32),
        mesh=_mesh(),
        scratch_types=[
            pltpu.VMEM((NCHUNK, CH), jnp.int32),
            pltpu.VMEM((NCHUNK, CH), jnp.int32),
            pltpu.VMEM((CH, HALF), jnp.float32),
            pltpu.VMEM_SHARED((NP, HALF), jnp.float32),
            pltpu.SemaphoreType.DMA,
        ],
    )


def _dot(a, b):
    return lax.dot_general(a, b, (((1,), (0,)), ((), ())),
                           precision=lax.Precision.HIGHEST,
                           preferred_element_type=jnp.float32)


def _layer_body(agg_ref, h_ref, deg_ref, wl_ref, bl_ref, wr_ref, out_ref):
    a = jnp.concatenate([agg_ref[0], agg_ref[1]], axis=1)
    hh = jnp.concatenate([h_ref[0], h_ref[1]], axis=1)
    rdeg = 1.0 / jnp.maximum(deg_ref[...], 1.0)
    scale = jnp.concatenate([rdeg, rdeg], axis=1)
    out = _dot(a * scale, wl_ref[...]) + _dot(hh, wr_ref[...]) + bl_ref[...]
    out = jnp.maximum(out, 0.0)
    out_ref[0] = out[:, :HALF]
    out_ref[1] = out[:, HALF:]


_ROWB = 256
_GRID = NP // _ROWB


def _tc_layer(agg, h, degb, wl, bl, wr):
    return pl.pallas_call(
        _layer_body,
        grid=(_GRID,),
        in_specs=[
            pl.BlockSpec((2, _ROWB, HALF), lambda i: (0, i, 0)),
            pl.BlockSpec((2, _ROWB, HALF), lambda i: (0, i, 0)),
            pl.BlockSpec((_ROWB, HALF), lambda i: (i, 0)),
            pl.BlockSpec((D, D), lambda i: (0, 0)),
            pl.BlockSpec((1, D), lambda i: (0, 0)),
            pl.BlockSpec((D, D), lambda i: (0, 0)),
        ],
        out_specs=pl.BlockSpec((2, _ROWB, HALF), lambda i: (0, i, 0)),
        out_shape=jax.ShapeDtypeStruct((2, NP, HALF), jnp.float32),
    )(agg, h, degb, wl, bl, wr)


def _pool_body(h_ref, b_ref, w1_ref, b1_ref, w2_ref, b2_ref, out_ref,
               acc_ref, cnt_ref):
    i = pl.program_id(0)

    @pl.when(i == 0)
    def _():
        acc_ref[...] = jnp.zeros_like(acc_ref)
        cnt_ref[...] = jnp.zeros_like(cnt_ref)

    hh = jnp.concatenate([h_ref[0], h_ref[1]], axis=1)
    gids = lax.broadcasted_iota(jnp.int32, (G, _ROWB), 0)
    mask = (jnp.broadcast_to(b_ref[0], (G, _ROWB)) == gids)
    mask = mask.astype(jnp.float32)
    acc_ref[...] += _dot(mask, hh)
    rs = jnp.sum(mask, axis=1, keepdims=True)
    cnt_ref[...] += jnp.broadcast_to(rs, (G, D))

    @pl.when(i == _GRID - 1)
    def _():
        pooled = acc_ref[...] / jnp.maximum(cnt_ref[...], 1.0)
        z = jnp.maximum(_dot(pooled, w1_ref[...]) + b1_ref[...], 0.0)
        logits = _dot(z, w2_ref[...]) + b2_ref[...]
        m = jnp.max(logits, axis=1, keepdims=True)
        lse = jnp.log(jnp.sum(jnp.exp(logits - m), axis=1, keepdims=True)) + m
        out_ref[...] = logits - lse


def _tc_pool_head(h, batch2d, w1, b1, w2p, b2p):
    return pl.pallas_call(
        _pool_body,
        grid=(_GRID,),
        in_specs=[
            pl.BlockSpec((2, _ROWB, HALF), lambda i: (0, i, 0)),
            pl.BlockSpec((1, 1, _ROWB), lambda i: (i, 0, 0)),
            pl.BlockSpec((D, D), lambda i: (0, 0)),
            pl.BlockSpec((1, D), lambda i: (0, 0)),
            pl.BlockSpec((D, HALF), lambda i: (0, 0)),
            pl.BlockSpec((1, HALF), lambda i: (0, 0)),
        ],
        out_specs=pl.BlockSpec((G, HALF), lambda i: (0, 0)),
        out_shape=jax.ShapeDtypeStruct((G, HALF), jnp.float32),
        scratch_shapes=[
            pltpu.VMEM((G, D), jnp.float32),
            pltpu.VMEM((G, D), jnp.float32),
        ],
    )(h, batch2d, w1, b1, w2p, b2p)


@jax.jit
def kernel(x, edge_index, batch, Wl1, bl1, Wr1, Wl2, bl2, Wr2, Wl3, bl3, Wr3,
           W_lin1, b_lin1, W_lin2, b_lin2):
    src = edge_index[0].astype(jnp.int32)
    dst = edge_index[1].astype(jnp.int32)
    src_t = jnp.concatenate(
        [src, jnp.zeros((EPAD - E,), jnp.int32)]).reshape(NSUB, NCHUNK, CH)
    dst_t = jnp.concatenate(
        [dst, jnp.full((EPAD - E,), N, jnp.int32)]).reshape(NSUB, NCHUNK, CH)

    deg = _sc_degree()(dst_t)
    degb = jnp.broadcast_to(deg[:, :1], (NP, HALF))

    xp = jnp.concatenate([x, jnp.zeros((NP - N, D), jnp.float32)])
    h = jnp.stack([xp[:, :HALF], xp[:, HALF:]])

    for wl, bl, wr in ((Wl1, bl1, Wr1), (Wl2, bl2, Wr2), (Wl3, bl3, Wr3)):
        agg = _sc_aggregate()(h.reshape(2 * NP, HALF), src_t, dst_t)
        h = _tc_layer(agg, h, degb, wl, bl.reshape(1, D), wr)

    batch2d = jnp.concatenate(
        [batch.astype(jnp.int32), jnp.full((NP - N,), G, jnp.int32)]
    ).reshape(_GRID, 1, _ROWB)
    w2p = jnp.zeros((D, HALF), jnp.float32).at[:, :NCLS].set(W_lin2)
    b2p = jnp.full((HALF,), -1e30, jnp.float32).at[:NCLS].set(b_lin2)

    out = _tc_pool_head(h, batch2d, W_lin1, b_lin1.reshape(1, D),
                        w2p, b2p.reshape(1, HALF))
    return out[:, :NCLS]
